# baseline (speedup 1.0000x reference)
import functools

import jax
import jax.numpy as jnp
from jax import lax
from jax.experimental import pallas as pl
from jax.experimental.pallas import tpu as pltpu
from jax.experimental.pallas import tpu_sc as plsc

N_NODES = 10000
N_EDGES = 320000
D = 128
DH = D // 2
NDEN = 640
CHUNK = 128
NCHUNKS = N_EDGES // CHUNK
N_PAD = 10240
ROWS_PER_TILE = N_PAD // 16


def _proj_body(h_ref, w_ref, al_ref, ar_ref, hp_ref, el_ref, er_ref):
    j = pl.program_id(1)
    hp = lax.dot_general(h_ref[...], w_ref[...], (((1,), (1,)), ((), ())),
                         preferred_element_type=jnp.float32)
    hp_ref[0] = hp
    el = lax.dot_general(hp, al_ref[0], (((1,), (1,)), ((), ())),
                         preferred_element_type=jnp.float32)
    er = lax.dot_general(hp, ar_ref[0], (((1,), (1,)), ((), ())),
                         preferred_element_type=jnp.float32)

    @pl.when(j == 0)
    def _():
        el_ref[...] = el
        er_ref[...] = er

    @pl.when(j != 0)
    def _():
        el_ref[...] += el
        er_ref[...] += er


_PROJ_ROWS = 1000


@jax.jit
def _proj(h, W, a_left, a_right):
    grid = (N_NODES // _PROJ_ROWS, 2)
    return pl.pallas_call(
        _proj_body,
        grid=grid,
        in_specs=[
            pl.BlockSpec((_PROJ_ROWS, D), lambda i, j: (i, 0)),
            pl.BlockSpec((DH, D), lambda i, j: (j, 0)),
            pl.BlockSpec((1, 1, DH), lambda i, j: (j, 0, 0)),
            pl.BlockSpec((1, 1, DH), lambda i, j: (j, 0, 0)),
        ],
        out_specs=[
            pl.BlockSpec((1, _PROJ_ROWS, DH), lambda i, j: (j, i, 0)),
            pl.BlockSpec((_PROJ_ROWS, 1), lambda i, j: (i, 0)),
            pl.BlockSpec((_PROJ_ROWS, 1), lambda i, j: (i, 0)),
        ],
        out_shape=[
            jax.ShapeDtypeStruct((2, N_NODES, DH), jnp.float32),
            jax.ShapeDtypeStruct((N_NODES, 1), jnp.float32),
            jax.ShapeDtypeStruct((N_NODES, 1), jnp.float32),
        ],
    )(h, W, a_left.reshape(2, 1, DH), a_right.reshape(2, 1, DH))


_MESH = plsc.VectorSubcoreMesh(core_axis_name="c", subcore_axis_name="s")


@functools.partial(
    pl.kernel,
    mesh=_MESH,
    out_type=[jax.ShapeDtypeStruct((2, N_PAD, DH), jnp.float32),
              jax.ShapeDtypeStruct((2, NDEN, 16), jnp.float32)],
    compiler_params=pltpu.CompilerParams(use_tc_tiling_on_sc=False,
                                         needs_layout_passes=False),
    scratch_types=[
        pltpu.VMEM((N_NODES,), jnp.float32),
        pltpu.VMEM((N_NODES,), jnp.float32),
        pltpu.VMEM((4, CHUNK), jnp.int32),
        pltpu.VMEM((4, CHUNK), jnp.int32),
        pltpu.VMEM((2, CHUNK), jnp.float32),
        pltpu.VMEM((2, CHUNK, DH), jnp.float32),
        pltpu.VMEM((2, CHUNK, DH), jnp.float32),
        pltpu.VMEM((NDEN, 16), jnp.float32),
        pltpu.VMEM((5, 128), jnp.int32),
        pltpu.VMEM_SHARED((N_PAD, DH), jnp.float32),
        pltpu.VMEM_SHARED((NDEN, 16), jnp.float32),
        pltpu.SemaphoreType.DMA,
        pltpu.SemaphoreType.DMA,
        pltpu.SemaphoreType.DMA,
        pltpu.SemaphoreType.DMA,
    ],
)
def _edge_kernel(hp_hbm, el_hbm, er_hbm, src_hbm, dst_hbm, iota_hbm,
                 out_hbm, den_hbm,
                 el_v, er_v, src_b, dst_b, w_b, rows_b, sc_b, denom_v, iv_v,
                 acc_sh, den_sh, sem_i, sem_g, sem_s, sem_d):
    cid = lax.axis_index("c")
    sid = lax.axis_index("s")

    pltpu.sync_copy(el_hbm, el_v)
    pltpu.sync_copy(er_hbm, er_v)
    pltpu.sync_copy(iota_hbm, iv_v)

    z16 = jnp.zeros((16,), jnp.float32)

    def zero_body(i, carry):
        for j in range(DH // 16):
            sc_b[0, i, pl.ds(j * 16, 16)] = z16
        return carry

    lax.fori_loop(0, CHUNK, zero_body, 0)

    @plsc.parallel_loop(0, NDEN, 1, unroll=8)
    def zero_den(i):
        denom_v[i, :] = z16

    for r in range(ROWS_PER_TILE // CHUNK):
        pltpu.sync_copy(sc_b.at[0],
                        acc_sh.at[pl.ds(sid * ROWS_PER_TILE + r * CHUNK, CHUNK)])
    pltpu.sync_copy(denom_v.at[pl.ds(sid * (NDEN // 16), NDEN // 16)],
                    den_sh.at[pl.ds(sid * (NDEN // 16), NDEN // 16)])
    plsc.subcore_barrier()

    nfull = NCHUNKS // 16
    nc = nfull + jnp.where(sid < NCHUNKS % 16, 1, 0)
    row_off = cid * N_NODES

    def idx_base(i):
        return (sid + i * 16) * CHUNK

    def issue_idx(i):
        ph = jnp.bitwise_and(i, 3)
        pltpu.async_copy(src_hbm.at[pl.ds(idx_base(i), CHUNK)],
                         src_b.at[ph], sem_i)
        pltpu.async_copy(dst_hbm.at[pl.ds(idx_base(i), CHUNK)],
                         dst_b.at[ph], sem_i)

    def wait_idx(i):
        ph = jnp.bitwise_and(i, 3)
        pltpu.make_async_copy(src_hbm.at[pl.ds(idx_base(i), CHUNK)],
                              src_b.at[ph], sem_i).wait()
        pltpu.make_async_copy(dst_hbm.at[pl.ds(idx_base(i), CHUNK)],
                              dst_b.at[ph], sem_i).wait()

    def wait_gather(i):
        ph2 = jnp.bitwise_and(i, 1)
        ph4 = jnp.bitwise_and(i, 3)
        pltpu.make_async_copy(hp_hbm.at[src_b.at[ph4]], rows_b.at[ph2],
                              sem_g).wait()

    def wait_scatter(i):
        ph2 = jnp.bitwise_and(i, 1)
        ph4 = jnp.bitwise_and(i, 3)
        pltpu.make_async_copy(sc_b.at[ph2], acc_sh.at[dst_b.at[ph4]],
                              sem_s).wait()

    issue_idx(0)

    def chunk_body(i, carry):
        @pl.when(i < nc)
        def _stage_x():
            ph2 = jnp.bitwise_and(i, 1)
            ph4 = jnp.bitwise_and(i, 3)
            wait_idx(i)
            for j in range(CHUNK // 16):
                s_ids = src_b[ph4, pl.ds(j * 16, 16)]
                d_ids = dst_b[ph4, pl.ds(j * 16, 16)]
                s = (plsc.load_gather(el_v, [s_ids])
                     + plsc.load_gather(er_v, [d_ids]))
                s = jnp.where(s > 0, s, 0.2 * s)
                w = jnp.exp(s)
                w_b[ph2, pl.ds(j * 16, 16)] = w
                src_b[ph4, pl.ds(j * 16, 16)] = s_ids + row_off
                d_rows = lax.shift_right_logical(d_ids, 4)
                d_cols = jnp.bitwise_and(d_ids, 15)
                plsc.addupdate_scatter(denom_v, [d_rows, d_cols], w)
            pltpu.async_copy(hp_hbm.at[src_b.at[ph4]], rows_b.at[ph2], sem_g)

            @pl.when(i + 1 < nc)
            def _():
                issue_idx(i + 1)

        @pl.when(i >= 1)
        def _stage_y():
            k_ = i - 1
            ph2 = jnp.bitwise_and(k_, 1)
            ph4 = jnp.bitwise_and(k_, 3)
            wait_gather(k_)

            @plsc.parallel_loop(0, CHUNK, 1, unroll=16)
            def edge_body(k):
                wk = plsc.load_gather(w_b.at[ph2],
                                      [jnp.zeros((16,), jnp.int32) + k])
                for j in range(DH // 16):
                    sc_b[ph2, k, pl.ds(j * 16, 16)] = (
                        rows_b[ph2, k, pl.ds(j * 16, 16)] * wk)

            @pl.when(i >= 3)
            def _():
                wait_scatter(i - 3)

            pltpu.async_copy(sc_b.at[ph2], acc_sh.at[dst_b.at[ph4]],
                             sem_s, add=True)

        return carry

    lax.fori_loop(0, nc + 1, chunk_body, 0)
    wait_scatter(nc - 1)
    wait_scatter(nc - 2)

    for r in range(5):
        pltpu.async_copy(denom_v.at[pl.ds(r * 128, 128)],
                         den_sh.at[iv_v.at[r]], sem_d, add=True)
    for r in range(5):
        pltpu.make_async_copy(denom_v.at[pl.ds(r * 128, 128)],
                              den_sh.at[iv_v.at[r]], sem_d).wait()

    plsc.subcore_barrier()
    rows = pl.ds(sid * ROWS_PER_TILE, ROWS_PER_TILE)
    pltpu.sync_copy(acc_sh.at[rows], out_hbm.at[cid].at[rows])
    drows = pl.ds(sid * (NDEN // 16), NDEN // 16)
    pltpu.sync_copy(den_sh.at[drows], den_hbm.at[cid].at[drows])


def _norm_body(p_ref, d_ref, o_ref):
    lo = p_ref[0]
    hi = p_ref[1]
    den_lo = d_ref[0]
    den_hi = d_ref[1]
    lo = jnp.where(den_lo > 0, lo / den_lo, 0.0)
    hi = jnp.where(den_hi > 0, hi / den_hi, 0.0)
    o_ref[...] = jnp.concatenate([lo, hi], axis=1)


@jax.jit
def _norm(p, den):
    grid = N_NODES // _PROJ_ROWS
    return pl.pallas_call(
        _norm_body,
        grid=(grid,),
        in_specs=[pl.BlockSpec((2, _PROJ_ROWS, DH), lambda i: (0, i, 0)),
                  pl.BlockSpec((2, _PROJ_ROWS, 1), lambda i: (0, i, 0))],
        out_specs=pl.BlockSpec((_PROJ_ROWS, D), lambda i: (i, 0)),
        out_shape=jax.ShapeDtypeStruct((N_NODES, D), jnp.float32),
    )(p, den)


@jax.jit
def kernel(h, edge_index, W, a_left, a_right):
    src = edge_index[0].astype(jnp.int32)
    dst = edge_index[1].astype(jnp.int32)
    hp, el, er = _proj(h, W, a_left, a_right)
    hp_flat = hp.reshape(2 * N_NODES, DH)
    iota = jnp.arange(NDEN, dtype=jnp.int32).reshape(5, 128)
    p, pd = _edge_kernel(hp_flat, el.reshape(N_NODES), er.reshape(N_NODES),
                         src, dst, iota)
    den = pd.reshape(2, N_PAD)[:, :N_NODES].reshape(2, N_NODES, 1)
    return _norm(p[:, :N_NODES], den)


# speedup vs baseline: 1.2342x; 1.2342x over previous
import functools

import jax
import jax.numpy as jnp
from jax import lax
from jax.experimental import pallas as pl
from jax.experimental.pallas import tpu as pltpu
from jax.experimental.pallas import tpu_sc as plsc

N_NODES = 10000
N_EDGES = 320000
D = 128
DH = D // 2
DW = DH + 16
CHUNK = 128
NCHUNKS = N_EDGES // CHUNK
N_PAD = 10240
ROWS_PER_TILE = N_PAD // 16


def _proj_body(h_ref, w_ref, al_ref, ar_ref, hp_ref, el_ref, er_ref):
    j = pl.program_id(1)
    hp = lax.dot_general(h_ref[...], w_ref[...], (((1,), (1,)), ((), ())),
                         preferred_element_type=jnp.float32)
    hp_ref[0] = hp
    el = lax.dot_general(al_ref[0], hp, (((1,), (1,)), ((), ())),
                         preferred_element_type=jnp.float32)
    er = lax.dot_general(ar_ref[0], hp, (((1,), (1,)), ((), ())),
                         preferred_element_type=jnp.float32)

    @pl.when(j == 0)
    def _():
        el_ref[...] = el
        er_ref[...] = er

    @pl.when(j != 0)
    def _():
        el_ref[...] += el
        er_ref[...] += er


_PROJ_ROWS = 1024


@jax.jit
def _proj(h, W, a_left, a_right):
    grid = (pl.cdiv(N_NODES, _PROJ_ROWS), 2)
    return pl.pallas_call(
        _proj_body,
        grid=grid,
        in_specs=[
            pl.BlockSpec((_PROJ_ROWS, D), lambda i, j: (i, 0)),
            pl.BlockSpec((DH, D), lambda i, j: (j, 0)),
            pl.BlockSpec((1, 1, DH), lambda i, j: (j, 0, 0)),
            pl.BlockSpec((1, 1, DH), lambda i, j: (j, 0, 0)),
        ],
        out_specs=[
            pl.BlockSpec((1, _PROJ_ROWS, DH), lambda i, j: (j, i, 0)),
            pl.BlockSpec((1, _PROJ_ROWS), lambda i, j: (0, i)),
            pl.BlockSpec((1, _PROJ_ROWS), lambda i, j: (0, i)),
        ],
        out_shape=[
            jax.ShapeDtypeStruct((2, N_NODES, DH), jnp.float32),
            jax.ShapeDtypeStruct((1, N_NODES), jnp.float32),
            jax.ShapeDtypeStruct((1, N_NODES), jnp.float32),
        ],
    )(h, W, a_left.reshape(2, 1, DH), a_right.reshape(2, 1, DH))


_MESH = plsc.VectorSubcoreMesh(core_axis_name="c", subcore_axis_name="s")


@functools.partial(
    pl.kernel,
    mesh=_MESH,
    out_type=jax.ShapeDtypeStruct((N_NODES, D), jnp.float32),
    compiler_params=pltpu.CompilerParams(use_tc_tiling_on_sc=False,
                                         needs_layout_passes=False),
    scratch_types=[
        pltpu.VMEM((N_NODES,), jnp.float32),
        pltpu.VMEM((N_NODES,), jnp.float32),
        pltpu.VMEM((4, CHUNK), jnp.int32),
        pltpu.VMEM((4, CHUNK), jnp.int32),
        pltpu.VMEM((2, CHUNK), jnp.float32),
        pltpu.VMEM((2, CHUNK, DH), jnp.float32),
        pltpu.VMEM((2, CHUNK, DW), jnp.float32),
        pltpu.VMEM_SHARED((N_PAD, DW), jnp.float32),
        pltpu.SemaphoreType.DMA,
        pltpu.SemaphoreType.DMA,
        pltpu.SemaphoreType.DMA,
    ],
)
def _edge_kernel(hp_hbm, el_hbm, er_hbm, src_hbm, dst_hbm, out_hbm,
                 el_v, er_v, src_b, dst_b, w_b, rows_b, sc_b, acc_sh,
                 sem_i, sem_g, sem_s):
    cid = lax.axis_index("c")
    sid = lax.axis_index("s")

    pltpu.sync_copy(el_hbm.at[0], el_v)
    pltpu.sync_copy(er_hbm.at[0], er_v)

    z16 = jnp.zeros((16,), jnp.float32)

    def zero_body(i, carry):
        for j in range(DW // 16):
            sc_b[0, i, pl.ds(j * 16, 16)] = z16
        return carry

    lax.fori_loop(0, CHUNK, zero_body, 0)
    for r in range(ROWS_PER_TILE // CHUNK):
        pltpu.sync_copy(sc_b.at[0],
                        acc_sh.at[pl.ds(sid * ROWS_PER_TILE + r * CHUNK, CHUNK)])
    plsc.subcore_barrier()

    nfull = NCHUNKS // 16
    nc = nfull + jnp.where(sid < NCHUNKS % 16, 1, 0)
    row_off = cid * N_NODES

    def idx_base(i):
        return (sid + i * 16) * CHUNK

    def issue_idx(i):
        ph = jnp.bitwise_and(i, 3)
        pltpu.async_copy(src_hbm.at[pl.ds(idx_base(i), CHUNK)],
                         src_b.at[ph], sem_i)
        pltpu.async_copy(dst_hbm.at[pl.ds(idx_base(i), CHUNK)],
                         dst_b.at[ph], sem_i)

    def wait_idx(i):
        ph = jnp.bitwise_and(i, 3)
        pltpu.make_async_copy(src_hbm.at[pl.ds(idx_base(i), CHUNK)],
                              src_b.at[ph], sem_i).wait()
        pltpu.make_async_copy(dst_hbm.at[pl.ds(idx_base(i), CHUNK)],
                              dst_b.at[ph], sem_i).wait()

    def wait_gather(i):
        ph2 = jnp.bitwise_and(i, 1)
        ph4 = jnp.bitwise_and(i, 3)
        pltpu.make_async_copy(hp_hbm.at[src_b.at[ph4]], rows_b.at[ph2],
                              sem_g).wait()

    def wait_scatter(i):
        ph2 = jnp.bitwise_and(i, 1)
        ph4 = jnp.bitwise_and(i, 3)
        pltpu.make_async_copy(sc_b.at[ph2], acc_sh.at[dst_b.at[ph4]],
                              sem_s).wait()

    issue_idx(0)

    def chunk_body(i, carry):
        @pl.when(i < nc)
        def _stage_x():
            ph2 = jnp.bitwise_and(i, 1)
            ph4 = jnp.bitwise_and(i, 3)
            wait_idx(i)
            for j in range(CHUNK // 16):
                s_ids = src_b[ph4, pl.ds(j * 16, 16)]
                d_ids = dst_b[ph4, pl.ds(j * 16, 16)]
                s = (plsc.load_gather(el_v, [s_ids])
                     + plsc.load_gather(er_v, [d_ids]))
                s = jnp.where(s > 0, s, 0.2 * s)
                w_b[ph2, pl.ds(j * 16, 16)] = jnp.exp(s)
                src_b[ph4, pl.ds(j * 16, 16)] = s_ids + row_off
            pltpu.async_copy(hp_hbm.at[src_b.at[ph4]], rows_b.at[ph2], sem_g)

            @pl.when(i + 1 < nc)
            def _():
                issue_idx(i + 1)

        @pl.when(i >= 1)
        def _stage_y():
            k_ = i - 1
            ph2 = jnp.bitwise_and(k_, 1)
            ph4 = jnp.bitwise_and(k_, 3)
            wait_gather(k_)

            @plsc.parallel_loop(0, CHUNK, 1, unroll=8)
            def edge_body(k):
                wk = plsc.load_gather(w_b.at[ph2],
                                      [jnp.zeros((16,), jnp.int32) + k])
                for j in range(DH // 16):
                    sc_b[ph2, k, pl.ds(j * 16, 16)] = (
                        rows_b[ph2, k, pl.ds(j * 16, 16)] * wk)
                sc_b[ph2, k, pl.ds(DH, 16)] = wk

            @pl.when(i >= 3)
            def _():
                wait_scatter(i - 3)

            pltpu.async_copy(sc_b.at[ph2], acc_sh.at[dst_b.at[ph4]],
                             sem_s, add=True)

        return carry

    lax.fori_loop(0, nc + 1, chunk_body, 0)
    wait_scatter(nc - 1)
    wait_scatter(nc - 2)

    plsc.subcore_barrier()

    iota16 = lax.iota(jnp.int32, 16)

    def norm_rows(row0, n):
        pltpu.sync_copy(acc_sh.at[pl.ds(row0, n)], sc_b.at[0].at[pl.ds(0, n)])

        @plsc.parallel_loop(0, n, 1, unroll=8)
        def nb_body(k):
            den = plsc.load_gather(sc_b.at[0],
                                   [jnp.zeros((16,), jnp.int32) + k,
                                    iota16 + DH])
            inv = jnp.where(den > 0, 1.0 / den, 0.0)
            for j in range(DH // 16):
                rows_b[0, k, pl.ds(j * 16, 16)] = (
                    sc_b[0, k, pl.ds(j * 16, 16)] * inv)

        pltpu.sync_copy(rows_b.at[0].at[pl.ds(0, n)],
                        out_hbm.at[pl.ds(row0, n), pl.ds(cid * DH, DH)])

    base_row = sid * ROWS_PER_TILE
    for r in range(ROWS_PER_TILE // CHUNK):
        row0 = base_row + r * CHUNK

        @pl.when(row0 + CHUNK <= N_NODES)
        def _full(row0=row0):
            norm_rows(row0, CHUNK)

        @pl.when(jnp.logical_and(row0 < N_NODES, row0 + CHUNK > N_NODES))
        def _tail(row0=row0):
            norm_rows(row0, N_NODES % CHUNK)


@jax.jit
def kernel(h, edge_index, W, a_left, a_right):
    src = edge_index[0].astype(jnp.int32)
    dst = edge_index[1].astype(jnp.int32)
    hp, el, er = _proj(h, W, a_left, a_right)
    hp_flat = hp.reshape(2 * N_NODES, DH)
    return _edge_kernel(hp_flat, el, er, src, dst)


# speedup vs baseline: 1.2565x; 1.0181x over previous
import functools

import jax
import jax.numpy as jnp
from jax import lax
from jax.experimental import pallas as pl
from jax.experimental.pallas import tpu as pltpu
from jax.experimental.pallas import tpu_sc as plsc

N_NODES = 10000
N_EDGES = 320000
D = 128
DH = D // 2
DW = DH + 16
SUB = 128
CHUNK = 256
NCHUNKS = N_EDGES // CHUNK
N_PAD = 10240
ROWS_PER_TILE = N_PAD // 16


def _proj_body(h_ref, w_ref, al_ref, ar_ref, hp_ref, el_ref, er_ref):
    j = pl.program_id(1)
    hp = lax.dot_general(h_ref[...], w_ref[...], (((1,), (1,)), ((), ())),
                         preferred_element_type=jnp.float32)
    hp_ref[0] = hp
    el = lax.dot_general(al_ref[0], hp, (((1,), (1,)), ((), ())),
                         preferred_element_type=jnp.float32)
    er = lax.dot_general(ar_ref[0], hp, (((1,), (1,)), ((), ())),
                         preferred_element_type=jnp.float32)

    @pl.when(j == 0)
    def _():
        el_ref[...] = el
        er_ref[...] = er

    @pl.when(j != 0)
    def _():
        el_ref[...] += el
        er_ref[...] += er


_PROJ_ROWS = 1024


@jax.jit
def _proj(h, W, a_left, a_right):
    grid = (pl.cdiv(N_NODES, _PROJ_ROWS), 2)
    return pl.pallas_call(
        _proj_body,
        grid=grid,
        in_specs=[
            pl.BlockSpec((_PROJ_ROWS, D), lambda i, j: (i, 0)),
            pl.BlockSpec((DH, D), lambda i, j: (j, 0)),
            pl.BlockSpec((1, 1, DH), lambda i, j: (j, 0, 0)),
            pl.BlockSpec((1, 1, DH), lambda i, j: (j, 0, 0)),
        ],
        out_specs=[
            pl.BlockSpec((1, _PROJ_ROWS, DH), lambda i, j: (j, i, 0)),
            pl.BlockSpec((1, _PROJ_ROWS), lambda i, j: (0, i)),
            pl.BlockSpec((1, _PROJ_ROWS), lambda i, j: (0, i)),
        ],
        out_shape=[
            jax.ShapeDtypeStruct((2, N_NODES, DH), jnp.float32),
            jax.ShapeDtypeStruct((1, N_NODES), jnp.float32),
            jax.ShapeDtypeStruct((1, N_NODES), jnp.float32),
        ],
    )(h, W, a_left.reshape(2, 1, DH), a_right.reshape(2, 1, DH))


_MESH = plsc.VectorSubcoreMesh(core_axis_name="c", subcore_axis_name="s")


@functools.partial(
    pl.kernel,
    mesh=_MESH,
    out_type=jax.ShapeDtypeStruct((N_NODES, D), jnp.float32),
    compiler_params=pltpu.CompilerParams(use_tc_tiling_on_sc=False,
                                         needs_layout_passes=False),
    scratch_types=[
        pltpu.VMEM((N_NODES,), jnp.float32),
        pltpu.VMEM((N_NODES,), jnp.float32),
        pltpu.VMEM((4, 2, SUB), jnp.int32),
        pltpu.VMEM((4, 2, SUB), jnp.int32),
        pltpu.VMEM((2, CHUNK), jnp.float32),
        pltpu.VMEM((2, CHUNK, DH), jnp.float32),
        pltpu.VMEM((2, CHUNK, 16), jnp.float32),
        pltpu.VMEM_SHARED((N_PAD, DH), jnp.float32),
        pltpu.VMEM_SHARED((N_PAD, 16), jnp.float32),
        pltpu.SemaphoreType.DMA,
        pltpu.SemaphoreType.DMA,
        pltpu.SemaphoreType.DMA,
    ],
)
def _edge_kernel(hp_hbm, el_hbm, er_hbm, src_hbm, dst_hbm, out_hbm,
                 el_v, er_v, src_b, dst_b, w_b, rows_b, wr_b, acc_sh, den_sh,
                 sem_i, sem_g, sem_s):
    cid = lax.axis_index("c")
    sid = lax.axis_index("s")

    pltpu.sync_copy(el_hbm.at[0], el_v)
    pltpu.sync_copy(er_hbm.at[0], er_v)

    z16 = jnp.zeros((16,), jnp.float32)

    def zero_body(i, carry):
        for j in range(DH // 16):
            rows_b[0, i, pl.ds(j * 16, 16)] = z16
        wr_b[0, i, :] = z16
        return carry

    lax.fori_loop(0, CHUNK, zero_body, 0)
    for r in range(ROWS_PER_TILE // CHUNK):
        base0 = sid * ROWS_PER_TILE + r * CHUNK
        pltpu.sync_copy(rows_b.at[0], acc_sh.at[pl.ds(base0, CHUNK)])
        pltpu.sync_copy(wr_b.at[0], den_sh.at[pl.ds(base0, CHUNK)])
    base0 = sid * ROWS_PER_TILE + 2 * CHUNK
    pltpu.sync_copy(rows_b.at[0].at[0:SUB], acc_sh.at[pl.ds(base0, SUB)])
    pltpu.sync_copy(wr_b.at[0].at[0:SUB], den_sh.at[pl.ds(base0, SUB)])
    plsc.subcore_barrier()

    nfull = NCHUNKS // 16
    nc = nfull + jnp.where(sid < NCHUNKS % 16, 1, 0)
    row_off = cid * N_NODES

    def idx_base(i):
        return (sid + i * 16) * CHUNK

    def issue_idx(i):
        ph = jnp.bitwise_and(i, 3)
        for hh in range(2):
            pltpu.async_copy(
                src_hbm.at[pl.ds(idx_base(i) + hh * SUB, SUB)],
                src_b.at[ph, hh], sem_i)
            pltpu.async_copy(
                dst_hbm.at[pl.ds(idx_base(i) + hh * SUB, SUB)],
                dst_b.at[ph, hh], sem_i)

    def wait_idx(i):
        ph = jnp.bitwise_and(i, 3)
        for hh in range(2):
            pltpu.make_async_copy(
                src_hbm.at[pl.ds(idx_base(i) + hh * SUB, SUB)],
                src_b.at[ph, hh], sem_i).wait()
            pltpu.make_async_copy(
                dst_hbm.at[pl.ds(idx_base(i) + hh * SUB, SUB)],
                dst_b.at[ph, hh], sem_i).wait()

    def issue_gather(i):
        ph2 = jnp.bitwise_and(i, 1)
        ph4 = jnp.bitwise_and(i, 3)
        for hh in range(2):
            pltpu.async_copy(hp_hbm.at[src_b.at[ph4, hh]],
                             rows_b.at[ph2].at[pl.ds(hh * SUB, SUB)], sem_g)

    def wait_gather(i):
        ph2 = jnp.bitwise_and(i, 1)
        ph4 = jnp.bitwise_and(i, 3)
        for hh in range(2):
            pltpu.make_async_copy(
                hp_hbm.at[src_b.at[ph4, hh]],
                rows_b.at[ph2].at[pl.ds(hh * SUB, SUB)], sem_g).wait()

    def issue_scatter(i):
        ph2 = jnp.bitwise_and(i, 1)
        ph4 = jnp.bitwise_and(i, 3)
        for hh in range(2):
            pltpu.async_copy(rows_b.at[ph2].at[pl.ds(hh * SUB, SUB)],
                             acc_sh.at[dst_b.at[ph4, hh]], sem_s, add=True)
            pltpu.async_copy(wr_b.at[ph2].at[pl.ds(hh * SUB, SUB)],
                             den_sh.at[dst_b.at[ph4, hh]], sem_s, add=True)

    def wait_scatter(i):
        ph2 = jnp.bitwise_and(i, 1)
        ph4 = jnp.bitwise_and(i, 3)
        for hh in range(2):
            pltpu.make_async_copy(rows_b.at[ph2].at[pl.ds(hh * SUB, SUB)],
                                  acc_sh.at[dst_b.at[ph4, hh]], sem_s).wait()
            pltpu.make_async_copy(wr_b.at[ph2].at[pl.ds(hh * SUB, SUB)],
                                  den_sh.at[dst_b.at[ph4, hh]], sem_s).wait()

    issue_idx(0)

    def chunk_body(i, carry):
        @pl.when(i < nc)
        def _stage_x():
            ph2 = jnp.bitwise_and(i, 1)
            ph4 = jnp.bitwise_and(i, 3)
            wait_idx(i)
            for hh in range(2):
                for j in range(SUB // 16):
                    s_ids = src_b[ph4, hh, pl.ds(j * 16, 16)]
                    d_ids = dst_b[ph4, hh, pl.ds(j * 16, 16)]
                    s = (plsc.load_gather(el_v, [s_ids])
                         + plsc.load_gather(er_v, [d_ids]))
                    s = jnp.where(s > 0, s, 0.2 * s)
                    w_b[ph2, pl.ds(hh * SUB + j * 16, 16)] = jnp.exp(s)
                    src_b[ph4, hh, pl.ds(j * 16, 16)] = s_ids + row_off
            @pl.when(i >= 2)
            def _():
                wait_scatter(i - 2)

            issue_gather(i)

            @pl.when(i + 1 < nc)
            def _():
                issue_idx(i + 1)

        @pl.when(i >= 1)
        def _stage_y():
            k_ = i - 1
            ph2 = jnp.bitwise_and(k_, 1)
            ph4 = jnp.bitwise_and(k_, 3)
            wait_gather(k_)

            @plsc.parallel_loop(0, CHUNK, 1, unroll=8)
            def edge_body(k):
                wk = plsc.load_gather(w_b.at[ph2],
                                      [jnp.zeros((16,), jnp.int32) + k])
                for j in range(DH // 16):
                    rows_b[ph2, k, pl.ds(j * 16, 16)] = (
                        rows_b[ph2, k, pl.ds(j * 16, 16)] * wk)
                wr_b[ph2, k, :] = wk

            issue_scatter(k_)

        return carry

    lax.fori_loop(0, nc + 1, chunk_body, 0)
    wait_scatter(nc - 1)
    wait_scatter(nc - 2)

    plsc.subcore_barrier()

    def norm_rows(row0, n):
        pltpu.sync_copy(acc_sh.at[pl.ds(row0, n)], rows_b.at[0].at[pl.ds(0, n)])
        pltpu.sync_copy(den_sh.at[pl.ds(row0, n)], wr_b.at[0].at[pl.ds(0, n)])

        @plsc.parallel_loop(0, n, 1, unroll=8)
        def nb_body(k):
            den = wr_b[0, k, :]
            inv = jnp.where(den > 0, 1.0 / den, 0.0)
            for j in range(DH // 16):
                rows_b[0, k, pl.ds(j * 16, 16)] = (
                    rows_b[0, k, pl.ds(j * 16, 16)] * inv)

        pltpu.sync_copy(rows_b.at[0].at[pl.ds(0, n)],
                        out_hbm.at[pl.ds(row0, n), pl.ds(cid * DH, DH)])

    base_row = sid * ROWS_PER_TILE
    for r in range(ROWS_PER_TILE // CHUNK):
        row0 = base_row + r * CHUNK

        @pl.when(row0 + CHUNK <= N_NODES)
        def _full(row0=row0):
            norm_rows(row0, CHUNK)

        @pl.when(jnp.logical_and(row0 < N_NODES, row0 + CHUNK > N_NODES))
        def _tail(row0=row0):
            norm_rows(row0, N_NODES % CHUNK)


@jax.jit
def kernel(h, edge_index, W, a_left, a_right):
    src = edge_index[0].astype(jnp.int32)
    dst = edge_index[1].astype(jnp.int32)
    hp, el, er = _proj(h, W, a_left, a_right)
    hp_flat = hp.reshape(2 * N_NODES, DH)
    return _edge_kernel(hp_flat, el, er, src, dst)


# speedup vs baseline: 1.3686x; 1.0893x over previous
import functools

import jax
import jax.numpy as jnp
from jax import lax
from jax.experimental import pallas as pl
from jax.experimental.pallas import tpu as pltpu
from jax.experimental.pallas import tpu_sc as plsc

N_NODES = 10000
N_EDGES = 320000
D = 128
DH = D // 2
DW = DH + 16
SUB = 128
CHUNK = 256
NCHUNKS = N_EDGES // CHUNK
N_PAD = 10240
ROWS_PER_TILE = N_PAD // 16


def _proj_body(h_ref, w_ref, al_ref, ar_ref, hp_ref, el_ref, er_ref):
    hp = lax.dot_general(h_ref[...], w_ref[...], (((1,), (1,)), ((), ())),
                         preferred_element_type=jnp.float32)
    hp_ref[...] = hp
    el_ref[...] = lax.dot_general(al_ref[...], hp, (((1,), (1,)), ((), ())),
                                  preferred_element_type=jnp.float32)
    er_ref[...] = lax.dot_general(ar_ref[...], hp, (((1,), (1,)), ((), ())),
                                  preferred_element_type=jnp.float32)


_PROJ_ROWS = 1024


@jax.jit
def _proj(h, W, a_left, a_right):
    grid = (pl.cdiv(N_NODES, _PROJ_ROWS),)
    return pl.pallas_call(
        _proj_body,
        grid=grid,
        in_specs=[
            pl.BlockSpec((_PROJ_ROWS, D), lambda i: (i, 0)),
            pl.BlockSpec((D, D), lambda i: (0, 0)),
            pl.BlockSpec((1, D), lambda i: (0, 0)),
            pl.BlockSpec((1, D), lambda i: (0, 0)),
        ],
        out_specs=[
            pl.BlockSpec((_PROJ_ROWS, D), lambda i: (i, 0)),
            pl.BlockSpec((1, _PROJ_ROWS), lambda i: (0, i)),
            pl.BlockSpec((1, _PROJ_ROWS), lambda i: (0, i)),
        ],
        out_shape=[
            jax.ShapeDtypeStruct((N_NODES, D), jnp.float32),
            jax.ShapeDtypeStruct((1, N_NODES), jnp.float32),
            jax.ShapeDtypeStruct((1, N_NODES), jnp.float32),
        ],
    )(h, W, a_left, a_right)


_MESH = plsc.VectorSubcoreMesh(core_axis_name="c", subcore_axis_name="s")


@functools.partial(
    pl.kernel,
    mesh=_MESH,
    out_type=jax.ShapeDtypeStruct((N_NODES, D), jnp.float32),
    compiler_params=pltpu.CompilerParams(use_tc_tiling_on_sc=False,
                                         needs_layout_passes=False),
    scratch_types=[
        pltpu.VMEM((N_NODES,), jnp.float32),
        pltpu.VMEM((N_NODES,), jnp.float32),
        pltpu.VMEM((4, 2, SUB), jnp.int32),
        pltpu.VMEM((4, 2, SUB), jnp.int32),
        pltpu.VMEM((2, CHUNK), jnp.float32),
        pltpu.VMEM((2, CHUNK, DH), jnp.float32),
        pltpu.VMEM((2, CHUNK, 16), jnp.float32),
        pltpu.VMEM_SHARED((N_PAD, DH), jnp.float32),
        pltpu.VMEM_SHARED((N_PAD, 16), jnp.float32),
        pltpu.SemaphoreType.DMA,
        pltpu.SemaphoreType.DMA,
        pltpu.SemaphoreType.DMA,
    ],
)
def _edge_kernel(hp_hbm, el_hbm, er_hbm, src_hbm, dst_hbm, out_hbm,
                 el_v, er_v, src_b, dst_b, w_b, rows_b, wr_b, acc_sh, den_sh,
                 sem_i, sem_g, sem_s):
    cid = lax.axis_index("c")
    sid = lax.axis_index("s")

    pltpu.sync_copy(el_hbm.at[0], el_v)
    pltpu.sync_copy(er_hbm.at[0], er_v)

    z16 = jnp.zeros((16,), jnp.float32)

    def zero_body(i, carry):
        for j in range(DH // 16):
            rows_b[0, i, pl.ds(j * 16, 16)] = z16
        wr_b[0, i, :] = z16
        return carry

    lax.fori_loop(0, CHUNK, zero_body, 0)
    for r in range(ROWS_PER_TILE // CHUNK):
        base0 = sid * ROWS_PER_TILE + r * CHUNK
        pltpu.sync_copy(rows_b.at[0], acc_sh.at[pl.ds(base0, CHUNK)])
        pltpu.sync_copy(wr_b.at[0], den_sh.at[pl.ds(base0, CHUNK)])
    base0 = sid * ROWS_PER_TILE + 2 * CHUNK
    pltpu.sync_copy(rows_b.at[0].at[0:SUB], acc_sh.at[pl.ds(base0, SUB)])
    pltpu.sync_copy(wr_b.at[0].at[0:SUB], den_sh.at[pl.ds(base0, SUB)])
    plsc.subcore_barrier()

    nfull = NCHUNKS // 16
    nc = nfull + jnp.where(sid < NCHUNKS % 16, 1, 0)

    def idx_base(i):
        return (sid + i * 16) * CHUNK

    def issue_idx(i):
        ph = jnp.bitwise_and(i, 3)
        for hh in range(2):
            pltpu.async_copy(
                src_hbm.at[pl.ds(idx_base(i) + hh * SUB, SUB)],
                src_b.at[ph, hh], sem_i)
            pltpu.async_copy(
                dst_hbm.at[pl.ds(idx_base(i) + hh * SUB, SUB)],
                dst_b.at[ph, hh], sem_i)

    def wait_idx(i):
        ph = jnp.bitwise_and(i, 3)
        for hh in range(2):
            pltpu.make_async_copy(
                src_hbm.at[pl.ds(idx_base(i) + hh * SUB, SUB)],
                src_b.at[ph, hh], sem_i).wait()
            pltpu.make_async_copy(
                dst_hbm.at[pl.ds(idx_base(i) + hh * SUB, SUB)],
                dst_b.at[ph, hh], sem_i).wait()

    def issue_gather(i):
        ph2 = jnp.bitwise_and(i, 1)
        ph4 = jnp.bitwise_and(i, 3)
        for hh in range(2):
            pltpu.async_copy(hp_hbm.at[src_b.at[ph4, hh]],
                             rows_b.at[ph2].at[pl.ds(hh * SUB, SUB)], sem_g)

    def wait_gather(i):
        ph2 = jnp.bitwise_and(i, 1)
        ph4 = jnp.bitwise_and(i, 3)
        for hh in range(2):
            pltpu.make_async_copy(
                hp_hbm.at[src_b.at[ph4, hh]],
                rows_b.at[ph2].at[pl.ds(hh * SUB, SUB)], sem_g).wait()

    def issue_scatter(i):
        ph2 = jnp.bitwise_and(i, 1)
        ph4 = jnp.bitwise_and(i, 3)
        for hh in range(2):
            pltpu.async_copy(rows_b.at[ph2].at[pl.ds(hh * SUB, SUB)],
                             acc_sh.at[dst_b.at[ph4, hh]], sem_s, add=True)
            pltpu.async_copy(wr_b.at[ph2].at[pl.ds(hh * SUB, SUB)],
                             den_sh.at[dst_b.at[ph4, hh]], sem_s, add=True)

    def wait_scatter(i):
        ph2 = jnp.bitwise_and(i, 1)
        ph4 = jnp.bitwise_and(i, 3)
        for hh in range(2):
            pltpu.make_async_copy(rows_b.at[ph2].at[pl.ds(hh * SUB, SUB)],
                                  acc_sh.at[dst_b.at[ph4, hh]], sem_s).wait()
            pltpu.make_async_copy(wr_b.at[ph2].at[pl.ds(hh * SUB, SUB)],
                                  den_sh.at[dst_b.at[ph4, hh]], sem_s).wait()

    issue_idx(0)

    def chunk_body(i, carry):
        @pl.when(i < nc)
        def _stage_x():
            ph2 = jnp.bitwise_and(i, 1)
            ph4 = jnp.bitwise_and(i, 3)
            wait_idx(i)
            for hh in range(2):
                for j in range(SUB // 16):
                    s_ids = src_b[ph4, hh, pl.ds(j * 16, 16)]
                    d_ids = dst_b[ph4, hh, pl.ds(j * 16, 16)]
                    s = (plsc.load_gather(el_v, [s_ids])
                         + plsc.load_gather(er_v, [d_ids]))
                    s = jnp.where(s > 0, s, 0.2 * s)
                    w_b[ph2, pl.ds(hh * SUB + j * 16, 16)] = jnp.exp(s)
                    src_b[ph4, hh, pl.ds(j * 16, 16)] = s_ids + s_ids + cid
            @pl.when(i >= 2)
            def _():
                wait_scatter(i - 2)

            issue_gather(i)

            @pl.when(i + 1 < nc)
            def _():
                issue_idx(i + 1)

        @pl.when(i >= 1)
        def _stage_y():
            k_ = i - 1
            ph2 = jnp.bitwise_and(k_, 1)
            ph4 = jnp.bitwise_and(k_, 3)
            wait_gather(k_)

            @plsc.parallel_loop(0, CHUNK, 1, unroll=8)
            def edge_body(k):
                wk = plsc.load_gather(w_b.at[ph2],
                                      [jnp.zeros((16,), jnp.int32) + k])
                for j in range(DH // 16):
                    rows_b[ph2, k, pl.ds(j * 16, 16)] = (
                        rows_b[ph2, k, pl.ds(j * 16, 16)] * wk)
                wr_b[ph2, k, :] = wk

            issue_scatter(k_)

        return carry

    lax.fori_loop(0, nc + 1, chunk_body, 0)
    wait_scatter(nc - 1)
    wait_scatter(nc - 2)

    plsc.subcore_barrier()

    def norm_rows(row0, n):
        pltpu.sync_copy(acc_sh.at[pl.ds(row0, n)], rows_b.at[0].at[pl.ds(0, n)])
        pltpu.sync_copy(den_sh.at[pl.ds(row0, n)], wr_b.at[0].at[pl.ds(0, n)])

        @plsc.parallel_loop(0, n, 1, unroll=8)
        def nb_body(k):
            den = wr_b[0, k, :]
            inv = jnp.where(den > 0, 1.0 / den, 0.0)
            for j in range(DH // 16):
                rows_b[0, k, pl.ds(j * 16, 16)] = (
                    rows_b[0, k, pl.ds(j * 16, 16)] * inv)

        pltpu.sync_copy(rows_b.at[0].at[pl.ds(0, n)],
                        out_hbm.at[pl.ds(row0, n), pl.ds(cid * DH, DH)])

    base_row = sid * ROWS_PER_TILE
    for r in range(ROWS_PER_TILE // CHUNK):
        row0 = base_row + r * CHUNK

        @pl.when(row0 + CHUNK <= N_NODES)
        def _full(row0=row0):
            norm_rows(row0, CHUNK)

        @pl.when(jnp.logical_and(row0 < N_NODES, row0 + CHUNK > N_NODES))
        def _tail(row0=row0):
            norm_rows(row0, N_NODES % CHUNK)


@jax.jit
def kernel(h, edge_index, W, a_left, a_right):
    src = edge_index[0].astype(jnp.int32)
    dst = edge_index[1].astype(jnp.int32)
    hp, el, er = _proj(h, W, a_left, a_right)
    hp_flat = hp.reshape(2 * N_NODES, DH)
    return _edge_kernel(hp_flat, el, er, src, dst)
